# baseline (speedup 1.0000x reference)
import functools

import jax
import jax.numpy as jnp
from jax.experimental import pallas as pl


RATIO = 0.5

_NP = 10240
_BI = 256
_BR = 256
_JC = 2048


def _rank_kernel(score_col_ref, score_row_ref, rank_ref):
    i0 = pl.program_id(0) * _BI
    si = score_col_ref[...]
    sj = score_row_ref[0:1, :]
    gt = sj > si
    eq = sj == si
    jj = jax.lax.broadcasted_iota(jnp.int32, (_BI, _NP), 1)
    ii = i0 + jax.lax.broadcasted_iota(jnp.int32, (_BI, _NP), 0)
    wins = gt | (eq & (jj < ii))
    cnt = jnp.sum(jnp.where(wins, 1.0, 0.0), axis=1, keepdims=True)
    rank_ref[...] = cnt.astype(jnp.int32)


def _gather_kernel(rank_row_ref, xs_ref, out_ref):
    r0 = pl.program_id(0) * _BR
    rr = r0 + jax.lax.broadcasted_iota(jnp.int32, (_BR, _JC), 0)

    def body(c, acc):
        rkc = rank_row_ref[0:1, pl.ds(c * _JC, _JC)]
        p = jnp.where(rkc == rr, 1.0, 0.0)
        xc = xs_ref[pl.ds(c * _JC, _JC), :]
        return acc + jnp.dot(p, xc, precision=jax.lax.Precision.HIGHEST,
                             preferred_element_type=jnp.float32)

    acc = jax.lax.fori_loop(0, _NP // _JC, body,
                            jnp.zeros((_BR, xs_ref.shape[1]), jnp.float32))
    out_ref[...] = acc


def _topk_gather(score, x, n_nodes, k):
    score_pad = jnp.full((_NP,), -2.0, jnp.float32).at[:n_nodes].set(score)
    score_col = score_pad.reshape(_NP, 1)
    score_row = jnp.broadcast_to(score_pad.reshape(1, _NP), (8, _NP))

    rank = pl.pallas_call(
        _rank_kernel,
        grid=(_NP // _BI,),
        in_specs=[
            pl.BlockSpec((_BI, 1), lambda i: (i, 0)),
            pl.BlockSpec((8, _NP), lambda i: (0, 0)),
        ],
        out_specs=pl.BlockSpec((_BI, 1), lambda i: (i, 0)),
        out_shape=jax.ShapeDtypeStruct((_NP, 1), jnp.int32),
    )(score_col, score_row)

    d = x.shape[1]
    cp = 384
    xs = jnp.zeros((_NP, cp), jnp.float32)
    xs = xs.at[:n_nodes, :d].set(x)
    xs = xs.at[:n_nodes, d].set(score)

    rank_row = jnp.broadcast_to(rank.reshape(1, _NP), (8, _NP))
    kp = 5120
    out = pl.pallas_call(
        _gather_kernel,
        grid=(kp // _BR,),
        in_specs=[
            pl.BlockSpec((8, _NP), lambda r: (0, 0)),
            pl.BlockSpec((_NP, cp), lambda r: (0, 0)),
        ],
        out_specs=pl.BlockSpec((_BR, cp), lambda r: (r, 0)),
        out_shape=jax.ShapeDtypeStruct((kp, cp), jnp.float32),
    )(rank_row, xs)

    xg = out[:k, :d]
    sg = out[:k, d:d + 1]
    return xg * sg


def kernel(x, edge_index, batch, W_l, b_l, W_r):
    n = x.shape[0]
    row, col = edge_index[0], edge_index[1]

    agg = jnp.zeros_like(x).at[col].add(x[row])
    score = (agg @ W_l.T + b_l + x @ W_r.T).reshape(-1)
    score = jnp.tanh(score)

    k = (n + 1) // 2
    x_out = _topk_gather(score, x, n, k)
    batch_out = jnp.zeros((k,), jnp.int32)
    return (x_out, batch_out)


# speedup vs baseline: 1.2797x; 1.2797x over previous
import functools

import jax
import jax.numpy as jnp
from jax import lax
from jax.experimental import pallas as pl
from jax.experimental.pallas import tpu as pltpu
from jax.experimental.pallas import tpu_sc as plsc


RATIO = 0.5

_NP = 10240
_NW = 32
_NPW = 320
_D = 256
_BI = 256


def _rank_kernel(score_col_ref, score_row_ref, rank_ref):
    i0 = pl.program_id(0) * _BI
    si = score_col_ref[...]
    sj = score_row_ref[0:1, :]
    gt = sj > si
    eq = sj == si
    jj = lax.broadcasted_iota(jnp.int32, (_BI, _NP), 1)
    ii = i0 + lax.broadcasted_iota(jnp.int32, (_BI, _NP), 0)
    wins = gt | (eq & (jj < ii))
    cnt = jnp.sum(jnp.where(wins, 1.0, 0.0), axis=1, keepdims=True)
    rank_ref[...] = cnt.astype(jnp.int32)


def _scatter_body(rank_hbm, xs_hbm, out_hbm, idx_v, xr_v, sem):
    wid = lax.axis_index("s") * 2 + lax.axis_index("c")
    pltpu.sync_copy(rank_hbm.at[wid], idx_v)
    pltpu.sync_copy(xs_hbm.at[pl.ds(wid * _NPW, _NPW), :], xr_v)
    pltpu.async_copy(xr_v, out_hbm.at[idx_v], sem).wait()


_scatter_out = functools.partial(
    pl.kernel,
    mesh=plsc.VectorSubcoreMesh(core_axis_name="c", subcore_axis_name="s"),
    out_type=jax.ShapeDtypeStruct((_NP, _D), jnp.float32),
    scratch_types=[
        pltpu.VMEM((_NPW,), jnp.int32),
        pltpu.VMEM((_NPW, _D), jnp.float32),
        pltpu.SemaphoreType.DMA,
    ],
)(_scatter_body)


def _topk_gather(score, x, n_nodes, k):
    score_pad = jnp.full((_NP,), -2.0, jnp.float32).at[:n_nodes].set(score)
    score_col = score_pad.reshape(_NP, 1)
    score_row = jnp.broadcast_to(score_pad.reshape(1, _NP), (8, _NP))

    rank = pl.pallas_call(
        _rank_kernel,
        grid=(_NP // _BI,),
        in_specs=[
            pl.BlockSpec((_BI, 1), lambda i: (i, 0)),
            pl.BlockSpec((8, _NP), lambda i: (0, 0)),
        ],
        out_specs=pl.BlockSpec((_BI, 1), lambda i: (i, 0)),
        out_shape=jax.ShapeDtypeStruct((_NP, 1), jnp.int32),
    )(score_col, score_row)

    xs = jnp.zeros((_NP, _D), jnp.float32).at[:n_nodes].set(x * score[:, None])
    out = _scatter_out(rank.reshape(_NW, _NPW), xs)
    return out[:k]


def kernel(x, edge_index, batch, W_l, b_l, W_r):
    n = x.shape[0]
    row, col = edge_index[0], edge_index[1]

    agg = jnp.zeros_like(x).at[col].add(x[row])
    score = (agg @ W_l.T + b_l + x @ W_r.T).reshape(-1)
    score = jnp.tanh(score)

    k = (n + 1) // 2
    x_out = _topk_gather(score, x, n, k)
    batch_out = jnp.zeros((k,), jnp.int32)
    return (x_out, batch_out)
